# baseline (speedup 1.0000x reference)
import functools
import math

import jax
import jax.numpy as jnp
from jax import lax
from jax.experimental import pallas as pl
from jax.experimental.pallas import tpu as pltpu
from jax.experimental.pallas import tpu_sc as plsc

_L = 2048
_FREQ = _L // 2 + 1
_FPAD = 1152
_TOPK = int(2 * math.log(_L))
_NROWS = 8192
_LANES = 16


def _split(x):
    hi = x.astype(jnp.bfloat16)
    lo = (x - hi.astype(jnp.float32)).astype(jnp.bfloat16)
    return hi, lo


def _mm3(a, b, dn):
    ah, al = _split(a)
    bh, bl = _split(b)
    f32 = jnp.float32
    return (lax.dot_general(ah, bh, dn, preferred_element_type=f32)
            + lax.dot_general(ah, bl, dn, preferred_element_type=f32)
            + lax.dot_general(al, bh, dn, preferred_element_type=f32))


def _mm3_pre(a, bh, bl, dn):
    ah, al = _split(a)
    f32 = jnp.float32
    return (lax.dot_general(ah, bh, dn, preferred_element_type=f32)
            + lax.dot_general(ah, bl, dn, preferred_element_type=f32)
            + lax.dot_general(al, bh, dn, preferred_element_type=f32))


def _proj_body(x_ref, w_ref, b_ref, o_ref):
    dn = (((1,), (0,)), ((), ()))
    o_ref[0] = lax.dot_general(
        x_ref[0].astype(jnp.bfloat16), w_ref[...].astype(jnp.bfloat16), dn,
        preferred_element_type=jnp.float32) + b_ref[...]


def _project(x, w, b):
    bsz, slen, d = x.shape
    do = w.shape[1]
    return pl.pallas_call(
        _proj_body,
        grid=(bsz, 2),
        in_specs=[
            pl.BlockSpec((1, slen // 2, d), lambda i, t: (i, t, 0)),
            pl.BlockSpec((d, do), lambda i, t: (0, 0)),
            pl.BlockSpec((1, do), lambda i, t: (0, 0)),
        ],
        out_specs=pl.BlockSpec((1, slen // 2, do), lambda i, t: (i, t, 0)),
        out_shape=jax.ShapeDtypeStruct((bsz, slen, do), jnp.float32),
    )(x, w, b.reshape(1, do))


def _corr_body(q_ref, k_ref, v_ref, ch_ref, cl_ref, sh_ref, sl_ref, a_ref,
               corr_ref, vt_ref):
    q = q_ref[...]
    k = k_ref[...]
    v = v_ref[...]
    ch, cl = ch_ref[...], cl_ref[...]
    sh, sl = sh_ref[...], sl_ref[...]
    dn = (((0,), (0,)), ((), ()))
    qa = _mm3_pre(q, ch, cl, dn)
    qb = _mm3_pre(q, sh, sl, dn)
    ka = _mm3_pre(k, ch, cl, dn)
    kb = _mm3_pre(k, sh, sl, dn)
    alpha = a_ref[...]
    pre = (qa * ka + qb * kb) * alpha
    pim = (qa * kb - qb * ka) * alpha
    dnf = (((1,), (1,)), ((), ()))
    corr = _mm3_pre(pre, ch, cl, dnf) - _mm3_pre(pim, sh, sl, dnf)
    corr_ref[...] = corr
    vt_ref[...] = v.T


def _corr_rows(q2, k2, v2, cm, sm, alpha, nc=128):
    grid = (_NROWS // nc,)
    blk_in = pl.BlockSpec((_L, nc), lambda t: (0, t))
    blk_const = lambda shape: pl.BlockSpec(shape, lambda t: (0, 0))
    blk_out = pl.BlockSpec((nc, _L), lambda t: (t, 0))
    ch, cl = _split(cm)
    sh, sl = _split(sm)
    return pl.pallas_call(
        _corr_body,
        grid=grid,
        in_specs=[
            blk_in, blk_in, blk_in,
            blk_const((_L, _FPAD)), blk_const((_L, _FPAD)),
            blk_const((_L, _FPAD)), blk_const((_L, _FPAD)),
            blk_const((1, _FPAD)),
        ],
        out_specs=[blk_out, blk_out],
        out_shape=[
            jax.ShapeDtypeStruct((_NROWS, _L), jnp.float32),
            jax.ShapeDtypeStruct((_NROWS, _L), jnp.float32),
        ],
    )(q2, k2, v2, ch, cl, sh, sl, alpha)


def _sc_row_compute(crow, v2, acc):
    lane = lax.iota(jnp.int32, _LANES)

    t0v, t0d = plsc.sort_key_val(crow[pl.ds(0, _LANES)], lane)

    def merge(c, carry):
        tv, td = carry
        ch = crow[pl.ds(c * _LANES, _LANES)]

        def do_merge(tv, td):
            cv, cd = plsc.sort_key_val(ch, lane + c * _LANES)
            cv = lax.rev(cv, (0,))
            cd = lax.rev(cd, (0,))
            keep = tv >= cv
            mv = jnp.where(keep, tv, cv)
            md = jnp.where(keep, td, cd)
            nv, nd = plsc.sort_key_val(mv, md)
            return nv, nd

        hits = plsc.all_reduce_population_count(ch > tv[0])
        return lax.cond(hits[0] > 0, do_merge, lambda tv, td: (tv, td),
                        tv, td)

    tv, td = lax.fori_loop(1, _L // _LANES, merge, (t0v, t0d))
    tv, td = plsc.sort_key_val(tv, td, descending=True)

    sel = lane < _TOPK
    vals = jnp.where(sel, tv, jnp.float32(-1e30))
    m = jnp.max(vals)
    e = jnp.where(sel, jnp.exp(vals - m), jnp.float32(0.0))
    w = e / jnp.sum(e)

    ds = [td[i] for i in range(_TOPK)]
    ws = [w[i] for i in range(_TOPK)]

    def agg(c, _):
        base = c * _LANES
        s = ws[0] * v2[pl.ds(base + ds[0], _LANES)]
        for i in range(1, _TOPK):
            s = s + ws[i] * v2[pl.ds(base + ds[i], _LANES)]
        acc[pl.ds(base, _LANES)] = s
        return 0

    lax.fori_loop(0, _L // _LANES, agg, 0)


def _sc_body(corr_hbm, vt_hbm, out_hbm,
             ca, cb, va, vb, oa, ob, sa, sb, soa, sob):
    nsub = 16
    ncore = 2
    wid = lax.axis_index("s") * ncore + lax.axis_index("c")
    rows_per = _NROWS // (nsub * ncore)
    base = wid * rows_per

    def start_in(r, cbuf, vbuf, sem):
        pltpu.async_copy(corr_hbm.at[r], cbuf, sem)
        pltpu.async_copy(vt_hbm.at[r], vbuf.at[pl.ds(0, _L)], sem)
        pltpu.async_copy(vt_hbm.at[r], vbuf.at[pl.ds(_L, _L)], sem)

    def wait_in(r, cbuf, vbuf, sem):
        pltpu.make_async_copy(corr_hbm.at[r], cbuf, sem).wait()
        pltpu.make_async_copy(vt_hbm.at[r], vbuf.at[pl.ds(0, _L)], sem).wait()
        pltpu.make_async_copy(vt_hbm.at[r], vbuf.at[pl.ds(_L, _L)], sem).wait()

    start_in(base, ca, va, sa)

    def pair(g, _):
        r0 = base + 2 * g
        r1 = r0 + 1
        start_in(r1, cb, vb, sb)
        wait_in(r0, ca, va, sa)

        @pl.when(g > 0)
        def _():
            pltpu.make_async_copy(oa, out_hbm.at[r0], soa).wait()

        _sc_row_compute(ca, va, oa)
        pltpu.async_copy(oa, out_hbm.at[r0], soa)

        @pl.when(g < rows_per // 2 - 1)
        def _():
            start_in(r0 + 2, ca, va, sa)

        wait_in(r1, cb, vb, sb)

        @pl.when(g > 0)
        def _():
            pltpu.make_async_copy(ob, out_hbm.at[r1], sob).wait()

        _sc_row_compute(cb, vb, ob)
        pltpu.async_copy(ob, out_hbm.at[r1], sob)
        return 0

    lax.fori_loop(0, rows_per // 2, pair, 0)
    pltpu.make_async_copy(oa, out_hbm.at[base], soa).wait()
    pltpu.make_async_copy(ob, out_hbm.at[base], sob).wait()


def _sc_topk_agg(corr_rows, vt_rows):
    mesh = plsc.VectorSubcoreMesh(core_axis_name="c", subcore_axis_name="s")
    fn = pl.kernel(
        _sc_body,
        out_type=jax.ShapeDtypeStruct((_NROWS, _L), jnp.float32),
        mesh=mesh,
        scratch_types=[
            pltpu.VMEM((_L,), jnp.float32),
            pltpu.VMEM((_L,), jnp.float32),
            pltpu.VMEM((2 * _L,), jnp.float32),
            pltpu.VMEM((2 * _L,), jnp.float32),
            pltpu.VMEM((_L,), jnp.float32),
            pltpu.VMEM((_L,), jnp.float32),
            pltpu.SemaphoreType.DMA,
            pltpu.SemaphoreType.DMA,
            pltpu.SemaphoreType.DMA,
            pltpu.SemaphoreType.DMA,
        ],
        compiler_params=pltpu.CompilerParams(needs_layout_passes=False),
    )
    return fn(corr_rows, vt_rows)


def _out_body(a_ref, w_ref, b_ref, o_ref):
    a = a_ref[...]
    o = lax.dot_general(
        a.astype(jnp.bfloat16), w_ref[...].astype(jnp.bfloat16),
        (((0,), (0,)), ((), ())),
        preferred_element_type=jnp.float32)
    o_ref[...] = o + b_ref[...]


def _out_proj(agg2, w_o, b_o, tb=256):
    nh, d = 16, 512
    out = pl.pallas_call(
        _out_body,
        grid=(nh, _L // tb),
        in_specs=[
            pl.BlockSpec((d, tb), lambda j, t: (j, t)),
            pl.BlockSpec((d, d), lambda j, t: (0, 0)),
            pl.BlockSpec((1, d), lambda j, t: (0, 0)),
        ],
        out_specs=pl.BlockSpec((tb, d), lambda j, t: (t, j)),
        out_shape=jax.ShapeDtypeStruct((_L, nh * d), jnp.float32),
    )(agg2, w_o, b_o.reshape(1, d))
    return out


def _dft_mats():
    i = jnp.arange(_L, dtype=jnp.int32)[:, None]
    f = jnp.arange(_FPAD, dtype=jnp.int32)[None, :]
    phase = jnp.mod(i * f, _L).astype(jnp.float32)
    theta = (2.0 * jnp.pi / _L) * phase
    live = (jnp.arange(_FPAD) < _FREQ)[None, :]
    cm = jnp.where(live, jnp.cos(theta), 0.0)
    sm = jnp.where(live, jnp.sin(theta), 0.0)
    fa = jnp.arange(_FPAD)
    alpha = jnp.where(
        (fa == 0) | (fa == _L // 2), 1.0 / _L,
        jnp.where(fa < _FREQ, 2.0 / _L, 0.0)).astype(jnp.float32)
    return cm, sm, alpha.reshape(1, _FPAD)


def kernel(queries, keys, values, W_q, b_q, W_k, b_k, W_v, b_v, W_o, b_o):
    bsz, slen, d = queries.shape
    qf = _project(queries, W_q, b_q)
    kf = _project(keys, W_k, b_k)
    vf = _project(values, W_v, b_v)
    q2 = qf.reshape(_L, 16 * d)
    k2 = kf.reshape(_L, 16 * d)
    v2 = vf.reshape(_L, 16 * d)
    cm, sm, alpha = _dft_mats()
    corr2, vt2 = _corr_rows(q2, k2, v2, cm, sm, alpha)
    agg = _sc_topk_agg(corr2, vt2)
    out2 = _out_proj(agg, W_o, b_o)
    return out2.reshape(bsz, slen, d)


# speedup vs baseline: 1.2688x; 1.2688x over previous
import functools
import math

import jax
import jax.numpy as jnp
from jax import lax
from jax.experimental import pallas as pl
from jax.experimental.pallas import tpu as pltpu
from jax.experimental.pallas import tpu_sc as plsc

_L = 2048
_FREQ = _L // 2 + 1
_FPAD = 1152
_TOPK = int(2 * math.log(_L))
_NROWS = 8192
_LANES = 16


def _split(x):
    hi = x.astype(jnp.bfloat16)
    lo = (x - hi.astype(jnp.float32)).astype(jnp.bfloat16)
    return hi, lo


def _mm3(a, b, dn):
    ah, al = _split(a)
    bh, bl = _split(b)
    f32 = jnp.float32
    return (lax.dot_general(ah, bh, dn, preferred_element_type=f32)
            + lax.dot_general(ah, bl, dn, preferred_element_type=f32)
            + lax.dot_general(al, bh, dn, preferred_element_type=f32))


def _mm3_pre(a, bh, bl, dn):
    ah, al = _split(a)
    f32 = jnp.float32
    return (lax.dot_general(ah, bh, dn, preferred_element_type=f32)
            + lax.dot_general(ah, bl, dn, preferred_element_type=f32)
            + lax.dot_general(al, bh, dn, preferred_element_type=f32))


def _proj_body(x_ref, w_ref, b_ref, o_ref):
    dn = (((1,), (0,)), ((), ()))
    o_ref[0] = lax.dot_general(
        x_ref[0].astype(jnp.bfloat16), w_ref[...].astype(jnp.bfloat16), dn,
        preferred_element_type=jnp.float32) + b_ref[...]


def _project(x, w, b):
    bsz, slen, d = x.shape
    do = w.shape[1]
    return pl.pallas_call(
        _proj_body,
        grid=(bsz, 2),
        in_specs=[
            pl.BlockSpec((1, slen // 2, d), lambda i, t: (i, t, 0)),
            pl.BlockSpec((d, do), lambda i, t: (0, 0)),
            pl.BlockSpec((1, do), lambda i, t: (0, 0)),
        ],
        out_specs=pl.BlockSpec((1, slen // 2, do), lambda i, t: (i, t, 0)),
        out_shape=jax.ShapeDtypeStruct((bsz, slen, do), jnp.float32),
    )(x, w, b.reshape(1, do))


def _corr_body(q_ref, k_ref, v_ref, ch_ref, cl_ref, sh_ref, sl_ref, a_ref,
               corr_ref, vt_ref):
    q = q_ref[...]
    k = k_ref[...]
    v = v_ref[...]
    ch, cl = ch_ref[...], cl_ref[...]
    sh, sl = sh_ref[...], sl_ref[...]
    dn = (((0,), (0,)), ((), ()))
    qa = _mm3_pre(q, ch, cl, dn)
    qb = _mm3_pre(q, sh, sl, dn)
    ka = _mm3_pre(k, ch, cl, dn)
    kb = _mm3_pre(k, sh, sl, dn)
    alpha = a_ref[...]
    pre = (qa * ka + qb * kb) * alpha
    pim = (qa * kb - qb * ka) * alpha
    dnf = (((1,), (1,)), ((), ()))
    corr = _mm3_pre(pre, ch, cl, dnf) - _mm3_pre(pim, sh, sl, dnf)
    corr_ref[...] = corr
    vt_ref[...] = v.T


def _corr_rows(q2, k2, v2, cm, sm, alpha, nc=128):
    grid = (_NROWS // nc,)
    blk_in = pl.BlockSpec((_L, nc), lambda t: (0, t))
    blk_const = lambda shape: pl.BlockSpec(shape, lambda t: (0, 0))
    blk_out = pl.BlockSpec((nc, _L), lambda t: (t, 0))
    ch, cl = _split(cm)
    sh, sl = _split(sm)
    return pl.pallas_call(
        _corr_body,
        grid=grid,
        in_specs=[
            blk_in, blk_in, blk_in,
            blk_const((_L, _FPAD)), blk_const((_L, _FPAD)),
            blk_const((_L, _FPAD)), blk_const((_L, _FPAD)),
            blk_const((1, _FPAD)),
        ],
        out_specs=[blk_out, blk_out],
        out_shape=[
            jax.ShapeDtypeStruct((_NROWS, _L), jnp.float32),
            jax.ShapeDtypeStruct((_NROWS, _L), jnp.float32),
        ],
    )(q2, k2, v2, ch, cl, sh, sl, alpha)


def _sc_row_compute(crow, v2, acc):
    lane = lax.iota(jnp.int32, _LANES)

    t0v, t0d = plsc.sort_key_val(crow[pl.ds(0, _LANES)], lane)

    def merge(c, carry):
        tv, td = carry
        ch = crow[pl.ds(c * _LANES, _LANES)]

        def do_merge(tv, td):
            cv, cd = plsc.sort_key_val(ch, lane + c * _LANES)
            cv = lax.rev(cv, (0,))
            cd = lax.rev(cd, (0,))
            keep = tv >= cv
            mv = jnp.where(keep, tv, cv)
            md = jnp.where(keep, td, cd)
            nv, nd = plsc.sort_key_val(mv, md)
            return nv, nd

        return do_merge(tv, td)

    tv, td = lax.fori_loop(1, _L // _LANES, merge, (t0v, t0d))
    tv, td = plsc.sort_key_val(tv, td, descending=True)

    sel = lane < _TOPK
    vals = jnp.where(sel, tv, jnp.float32(-1e30))
    m = jnp.max(vals)
    e = jnp.where(sel, jnp.exp(vals - m), jnp.float32(0.0))
    w = e / jnp.sum(e)

    ds = [td[i] for i in range(_TOPK)]
    ws = [w[i] for i in range(_TOPK)]

    def agg(c, _):
        base = c * _LANES
        s = ws[0] * v2[pl.ds(base + ds[0], _LANES)]
        for i in range(1, _TOPK):
            s = s + ws[i] * v2[pl.ds(base + ds[i], _LANES)]
        acc[pl.ds(base, _LANES)] = s
        return 0

    lax.fori_loop(0, _L // _LANES, agg, 0)


def _sc_body(corr_hbm, vt_hbm, out_hbm,
             ca, cb, va, vb, oa, ob, sa, sb, soa, sob):
    nsub = 16
    ncore = 2
    wid = lax.axis_index("s") * ncore + lax.axis_index("c")
    rows_per = _NROWS // (nsub * ncore)
    base = wid * rows_per

    def start_in(r, cbuf, vbuf, sem):
        pltpu.async_copy(corr_hbm.at[r], cbuf, sem)
        pltpu.async_copy(vt_hbm.at[r], vbuf.at[pl.ds(0, _L)], sem)
        pltpu.async_copy(vt_hbm.at[r], vbuf.at[pl.ds(_L, _L)], sem)

    def wait_in(r, cbuf, vbuf, sem):
        pltpu.make_async_copy(corr_hbm.at[r], cbuf, sem).wait()
        pltpu.make_async_copy(vt_hbm.at[r], vbuf.at[pl.ds(0, _L)], sem).wait()
        pltpu.make_async_copy(vt_hbm.at[r], vbuf.at[pl.ds(_L, _L)], sem).wait()

    start_in(base, ca, va, sa)

    def pair(g, _):
        r0 = base + 2 * g
        r1 = r0 + 1
        start_in(r1, cb, vb, sb)
        wait_in(r0, ca, va, sa)

        @pl.when(g > 0)
        def _():
            pltpu.make_async_copy(oa, out_hbm.at[r0], soa).wait()

        _sc_row_compute(ca, va, oa)
        pltpu.async_copy(oa, out_hbm.at[r0], soa)

        @pl.when(g < rows_per // 2 - 1)
        def _():
            start_in(r0 + 2, ca, va, sa)

        wait_in(r1, cb, vb, sb)

        @pl.when(g > 0)
        def _():
            pltpu.make_async_copy(ob, out_hbm.at[r1], sob).wait()

        _sc_row_compute(cb, vb, ob)
        pltpu.async_copy(ob, out_hbm.at[r1], sob)
        return 0

    lax.fori_loop(0, rows_per // 2, pair, 0)
    pltpu.make_async_copy(oa, out_hbm.at[base], soa).wait()
    pltpu.make_async_copy(ob, out_hbm.at[base], sob).wait()


def _sc_topk_agg(corr_rows, vt_rows):
    mesh = plsc.VectorSubcoreMesh(core_axis_name="c", subcore_axis_name="s")
    fn = pl.kernel(
        _sc_body,
        out_type=jax.ShapeDtypeStruct((_NROWS, _L), jnp.float32),
        mesh=mesh,
        scratch_types=[
            pltpu.VMEM((_L,), jnp.float32),
            pltpu.VMEM((_L,), jnp.float32),
            pltpu.VMEM((2 * _L,), jnp.float32),
            pltpu.VMEM((2 * _L,), jnp.float32),
            pltpu.VMEM((_L,), jnp.float32),
            pltpu.VMEM((_L,), jnp.float32),
            pltpu.SemaphoreType.DMA,
            pltpu.SemaphoreType.DMA,
            pltpu.SemaphoreType.DMA,
            pltpu.SemaphoreType.DMA,
        ],
        compiler_params=pltpu.CompilerParams(needs_layout_passes=False),
    )
    return fn(corr_rows, vt_rows)


def _out_body(a_ref, w_ref, b_ref, o_ref):
    a = a_ref[...]
    o = lax.dot_general(
        a.astype(jnp.bfloat16), w_ref[...].astype(jnp.bfloat16),
        (((0,), (0,)), ((), ())),
        preferred_element_type=jnp.float32)
    o_ref[...] = o + b_ref[...]


def _out_proj(agg2, w_o, b_o, tb=256):
    nh, d = 16, 512
    out = pl.pallas_call(
        _out_body,
        grid=(nh, _L // tb),
        in_specs=[
            pl.BlockSpec((d, tb), lambda j, t: (j, t)),
            pl.BlockSpec((d, d), lambda j, t: (0, 0)),
            pl.BlockSpec((1, d), lambda j, t: (0, 0)),
        ],
        out_specs=pl.BlockSpec((tb, d), lambda j, t: (t, j)),
        out_shape=jax.ShapeDtypeStruct((_L, nh * d), jnp.float32),
    )(agg2, w_o, b_o.reshape(1, d))
    return out


def _dft_mats():
    i = jnp.arange(_L, dtype=jnp.int32)[:, None]
    f = jnp.arange(_FPAD, dtype=jnp.int32)[None, :]
    phase = jnp.mod(i * f, _L).astype(jnp.float32)
    theta = (2.0 * jnp.pi / _L) * phase
    live = (jnp.arange(_FPAD) < _FREQ)[None, :]
    cm = jnp.where(live, jnp.cos(theta), 0.0)
    sm = jnp.where(live, jnp.sin(theta), 0.0)
    fa = jnp.arange(_FPAD)
    alpha = jnp.where(
        (fa == 0) | (fa == _L // 2), 1.0 / _L,
        jnp.where(fa < _FREQ, 2.0 / _L, 0.0)).astype(jnp.float32)
    return cm, sm, alpha.reshape(1, _FPAD)


def kernel(queries, keys, values, W_q, b_q, W_k, b_k, W_v, b_v, W_o, b_o):
    bsz, slen, d = queries.shape
    qf = _project(queries, W_q, b_q)
    kf = _project(keys, W_k, b_k)
    vf = _project(values, W_v, b_v)
    q2 = qf.reshape(_L, 16 * d)
    k2 = kf.reshape(_L, 16 * d)
    v2 = vf.reshape(_L, 16 * d)
    cm, sm, alpha = _dft_mats()
    corr2, vt2 = _corr_rows(q2, k2, v2, cm, sm, alpha)
    agg = _sc_topk_agg(corr2, vt2)
    out2 = _out_proj(agg, W_o, b_o)
    return out2.reshape(bsz, slen, d)


# speedup vs baseline: 1.5470x; 1.2192x over previous
import functools
import math

import jax
import jax.numpy as jnp
from jax import lax
from jax.experimental import pallas as pl
from jax.experimental.pallas import tpu as pltpu
from jax.experimental.pallas import tpu_sc as plsc

_L = 2048
_FREQ = _L // 2 + 1
_FPAD = 1152
_TOPK = int(2 * math.log(_L))
_NROWS = 8192
_LANES = 16


def _split(x):
    hi = x.astype(jnp.bfloat16)
    lo = (x - hi.astype(jnp.float32)).astype(jnp.bfloat16)
    return hi, lo


def _mm3(a, b, dn):
    ah, al = _split(a)
    bh, bl = _split(b)
    f32 = jnp.float32
    return (lax.dot_general(ah, bh, dn, preferred_element_type=f32)
            + lax.dot_general(ah, bl, dn, preferred_element_type=f32)
            + lax.dot_general(al, bh, dn, preferred_element_type=f32))


def _mm3_pre(a, bh, bl, dn):
    ah, al = _split(a)
    f32 = jnp.float32
    return (lax.dot_general(ah, bh, dn, preferred_element_type=f32)
            + lax.dot_general(ah, bl, dn, preferred_element_type=f32)
            + lax.dot_general(al, bh, dn, preferred_element_type=f32))


def _proj_body(x_ref, w_ref, b_ref, o_ref):
    dn = (((1,), (0,)), ((), ()))
    o_ref[0] = lax.dot_general(
        x_ref[0].astype(jnp.bfloat16), w_ref[...].astype(jnp.bfloat16), dn,
        preferred_element_type=jnp.float32) + b_ref[...]


def _project(x, w, b):
    bsz, slen, d = x.shape
    do = w.shape[1]
    return pl.pallas_call(
        _proj_body,
        grid=(bsz, 2),
        in_specs=[
            pl.BlockSpec((1, slen // 2, d), lambda i, t: (i, t, 0)),
            pl.BlockSpec((d, do), lambda i, t: (0, 0)),
            pl.BlockSpec((1, do), lambda i, t: (0, 0)),
        ],
        out_specs=pl.BlockSpec((1, slen // 2, do), lambda i, t: (i, t, 0)),
        out_shape=jax.ShapeDtypeStruct((bsz, slen, do), jnp.float32),
    )(x, w, b.reshape(1, do))


def _corr_body(q_ref, k_ref, v_ref, ch_ref, cl_ref, sh_ref, sl_ref, a_ref,
               corr_ref, vt_ref):
    q = q_ref[...]
    k = k_ref[...]
    v = v_ref[...]
    ch, cl = ch_ref[...], cl_ref[...]
    sh, sl = sh_ref[...], sl_ref[...]
    dn = (((0,), (0,)), ((), ()))
    qa = _mm3_pre(q, ch, cl, dn)
    qb = _mm3_pre(q, sh, sl, dn)
    ka = _mm3_pre(k, ch, cl, dn)
    kb = _mm3_pre(k, sh, sl, dn)
    alpha = a_ref[...]
    pre = (qa * ka + qb * kb) * alpha
    pim = (qa * kb - qb * ka) * alpha
    dnf = (((1,), (1,)), ((), ()))
    corr = _mm3_pre(pre, ch, cl, dnf) - _mm3_pre(pim, sh, sl, dnf)
    corr_ref[...] = corr
    vt_ref[...] = v.T


def _corr_rows(q2, k2, v2, cm, sm, alpha, nc=256):
    grid = (_NROWS // nc,)
    blk_in = pl.BlockSpec((_L, nc), lambda t: (0, t))
    blk_const = lambda shape: pl.BlockSpec(shape, lambda t: (0, 0))
    blk_out = pl.BlockSpec((nc, _L), lambda t: (t, 0))
    ch, cl = _split(cm)
    sh, sl = _split(sm)
    return pl.pallas_call(
        _corr_body,
        grid=grid,
        in_specs=[
            blk_in, blk_in, blk_in,
            blk_const((_L, _FPAD)), blk_const((_L, _FPAD)),
            blk_const((_L, _FPAD)), blk_const((_L, _FPAD)),
            blk_const((1, _FPAD)),
        ],
        out_specs=[blk_out, blk_out],
        out_shape=[
            jax.ShapeDtypeStruct((_NROWS, _L), jnp.float32),
            jax.ShapeDtypeStruct((_NROWS, _L), jnp.float32),
        ],
    )(q2, k2, v2, ch, cl, sh, sl, alpha)


def _sc_row_compute(crow, v2, acc):
    lane = lax.iota(jnp.int32, _LANES)

    t0v, t0d = plsc.sort_key_val(crow[pl.ds(0, _LANES)], lane)

    def _top16(av, ad, bv, bd):
        bv = lax.rev(bv, (0,))
        bd = lax.rev(bd, (0,))
        keep = av >= bv
        mv = jnp.where(keep, av, bv)
        md = jnp.where(keep, ad, bd)
        nv, nd = plsc.sort_key_val(mv, md)
        return nv, nd

    def merge_pair(c, carry):
        tv, td = carry
        i0 = 2 * c + 1
        av, ad = plsc.sort_key_val(crow[pl.ds(i0 * _LANES, _LANES)],
                                   lane + i0 * _LANES)
        bv, bd = plsc.sort_key_val(crow[pl.ds((i0 + 1) * _LANES, _LANES)],
                                   lane + (i0 + 1) * _LANES)
        pv, pd = _top16(av, ad, bv, bd)
        return _top16(tv, td, pv, pd)

    npair = (_L // _LANES - 1) // 2
    tv, td = lax.fori_loop(0, npair, merge_pair, (t0v, t0d))
    lastc = _L // _LANES - 1
    lv, ld = plsc.sort_key_val(crow[pl.ds(lastc * _LANES, _LANES)],
                               lane + lastc * _LANES)
    tv, td = _top16(tv, td, lv, ld)
    tv, td = plsc.sort_key_val(tv, td, descending=True)

    sel = lane < _TOPK
    vals = jnp.where(sel, tv, jnp.float32(-1e30))
    m = jnp.max(vals)
    e = jnp.where(sel, jnp.exp(vals - m), jnp.float32(0.0))
    w = e / jnp.sum(e)

    ds = [td[i] for i in range(_TOPK)]
    ws = [w[i] for i in range(_TOPK)]

    def agg(c, _):
        base = c * _LANES
        s = ws[0] * v2[pl.ds(base + ds[0], _LANES)]
        for i in range(1, _TOPK):
            s = s + ws[i] * v2[pl.ds(base + ds[i], _LANES)]
        acc[pl.ds(base, _LANES)] = s
        return 0

    lax.fori_loop(0, _L // _LANES, agg, 0)


def _sc_body(corr_hbm, vt_hbm, out_hbm,
             ca, cb, va, vb, oa, ob, sa, sb, soa, sob):
    nsub = 16
    ncore = 2
    wid = lax.axis_index("s") * ncore + lax.axis_index("c")
    rows_per = _NROWS // (nsub * ncore)
    base = wid * rows_per

    def start_in(r, cbuf, vbuf, sem):
        pltpu.async_copy(corr_hbm.at[r], cbuf, sem)
        pltpu.async_copy(vt_hbm.at[r], vbuf.at[pl.ds(0, _L)], sem)
        pltpu.async_copy(vt_hbm.at[r], vbuf.at[pl.ds(_L, _L)], sem)

    def wait_in(r, cbuf, vbuf, sem):
        pltpu.make_async_copy(corr_hbm.at[r], cbuf, sem).wait()
        pltpu.make_async_copy(vt_hbm.at[r], vbuf.at[pl.ds(0, _L)], sem).wait()
        pltpu.make_async_copy(vt_hbm.at[r], vbuf.at[pl.ds(_L, _L)], sem).wait()

    start_in(base, ca, va, sa)

    def pair(g, _):
        r0 = base + 2 * g
        r1 = r0 + 1
        start_in(r1, cb, vb, sb)
        wait_in(r0, ca, va, sa)

        @pl.when(g > 0)
        def _():
            pltpu.make_async_copy(oa, out_hbm.at[r0], soa).wait()

        _sc_row_compute(ca, va, oa)
        pltpu.async_copy(oa, out_hbm.at[r0], soa)

        @pl.when(g < rows_per // 2 - 1)
        def _():
            start_in(r0 + 2, ca, va, sa)

        wait_in(r1, cb, vb, sb)

        @pl.when(g > 0)
        def _():
            pltpu.make_async_copy(ob, out_hbm.at[r1], sob).wait()

        _sc_row_compute(cb, vb, ob)
        pltpu.async_copy(ob, out_hbm.at[r1], sob)
        return 0

    lax.fori_loop(0, rows_per // 2, pair, 0)
    pltpu.make_async_copy(oa, out_hbm.at[base], soa).wait()
    pltpu.make_async_copy(ob, out_hbm.at[base], sob).wait()


def _sc_topk_agg(corr_rows, vt_rows):
    mesh = plsc.VectorSubcoreMesh(core_axis_name="c", subcore_axis_name="s")
    fn = pl.kernel(
        _sc_body,
        out_type=jax.ShapeDtypeStruct((_NROWS, _L), jnp.float32),
        mesh=mesh,
        scratch_types=[
            pltpu.VMEM((_L,), jnp.float32),
            pltpu.VMEM((_L,), jnp.float32),
            pltpu.VMEM((2 * _L,), jnp.float32),
            pltpu.VMEM((2 * _L,), jnp.float32),
            pltpu.VMEM((_L,), jnp.float32),
            pltpu.VMEM((_L,), jnp.float32),
            pltpu.SemaphoreType.DMA,
            pltpu.SemaphoreType.DMA,
            pltpu.SemaphoreType.DMA,
            pltpu.SemaphoreType.DMA,
        ],
        compiler_params=pltpu.CompilerParams(needs_layout_passes=False),
    )
    return fn(corr_rows, vt_rows)


def _out_body(a_ref, w_ref, b_ref, o_ref):
    a = a_ref[...]
    o = lax.dot_general(
        a.astype(jnp.bfloat16), w_ref[...].astype(jnp.bfloat16),
        (((0,), (0,)), ((), ())),
        preferred_element_type=jnp.float32)
    o_ref[...] = o + b_ref[...]


def _out_proj(agg2, w_o, b_o, tb=256):
    nh, d = 16, 512
    out = pl.pallas_call(
        _out_body,
        grid=(nh, _L // tb),
        in_specs=[
            pl.BlockSpec((d, tb), lambda j, t: (j, t)),
            pl.BlockSpec((d, d), lambda j, t: (0, 0)),
            pl.BlockSpec((1, d), lambda j, t: (0, 0)),
        ],
        out_specs=pl.BlockSpec((tb, d), lambda j, t: (t, j)),
        out_shape=jax.ShapeDtypeStruct((_L, nh * d), jnp.float32),
    )(agg2, w_o, b_o.reshape(1, d))
    return out


def _dft_mats():
    i = jnp.arange(_L, dtype=jnp.int32)[:, None]
    f = jnp.arange(_FPAD, dtype=jnp.int32)[None, :]
    phase = jnp.mod(i * f, _L).astype(jnp.float32)
    theta = (2.0 * jnp.pi / _L) * phase
    live = (jnp.arange(_FPAD) < _FREQ)[None, :]
    cm = jnp.where(live, jnp.cos(theta), 0.0)
    sm = jnp.where(live, jnp.sin(theta), 0.0)
    fa = jnp.arange(_FPAD)
    alpha = jnp.where(
        (fa == 0) | (fa == _L // 2), 1.0 / _L,
        jnp.where(fa < _FREQ, 2.0 / _L, 0.0)).astype(jnp.float32)
    return cm, sm, alpha.reshape(1, _FPAD)


def kernel(queries, keys, values, W_q, b_q, W_k, b_k, W_v, b_v, W_o, b_o):
    bsz, slen, d = queries.shape
    qf = _project(queries, W_q, b_q)
    kf = _project(keys, W_k, b_k)
    vf = _project(values, W_v, b_v)
    q2 = qf.reshape(_L, 16 * d)
    k2 = kf.reshape(_L, 16 * d)
    v2 = vf.reshape(_L, 16 * d)
    cm, sm, alpha = _dft_mats()
    corr2, vt2 = _corr_rows(q2, k2, v2, cm, sm, alpha)
    agg = _sc_topk_agg(corr2, vt2)
    out2 = _out_proj(agg, W_o, b_o)
    return out2.reshape(bsz, slen, d)


# speedup vs baseline: 2.0144x; 1.3021x over previous
import functools
import math

import jax
import jax.numpy as jnp
from jax import lax
from jax.experimental import pallas as pl
from jax.experimental.pallas import tpu as pltpu
from jax.experimental.pallas import tpu_sc as plsc

_L = 2048
_FREQ = _L // 2 + 1
_FPAD = 1152
_TOPK = int(2 * math.log(_L))
_NROWS = 8192
_LANES = 16


def _split(x):
    hi = x.astype(jnp.bfloat16)
    lo = (x - hi.astype(jnp.float32)).astype(jnp.bfloat16)
    return hi, lo


def _mm3(a, b, dn):
    ah, al = _split(a)
    bh, bl = _split(b)
    f32 = jnp.float32
    return (lax.dot_general(ah, bh, dn, preferred_element_type=f32)
            + lax.dot_general(ah, bl, dn, preferred_element_type=f32)
            + lax.dot_general(al, bh, dn, preferred_element_type=f32))


def _mm3_pre(a, bh, bl, dn):
    ah, al = _split(a)
    f32 = jnp.float32
    return (lax.dot_general(ah, bh, dn, preferred_element_type=f32)
            + lax.dot_general(ah, bl, dn, preferred_element_type=f32)
            + lax.dot_general(al, bh, dn, preferred_element_type=f32))


def _proj_body(x_ref, w_ref, b_ref, o_ref):
    dn = (((1,), (0,)), ((), ()))
    o_ref[0] = lax.dot_general(
        x_ref[0].astype(jnp.bfloat16), w_ref[...].astype(jnp.bfloat16), dn,
        preferred_element_type=jnp.float32) + b_ref[...]


def _project(x, w, b):
    bsz, slen, d = x.shape
    do = w.shape[1]
    return pl.pallas_call(
        _proj_body,
        grid=(bsz, 2),
        in_specs=[
            pl.BlockSpec((1, slen // 2, d), lambda i, t: (i, t, 0)),
            pl.BlockSpec((d, do), lambda i, t: (0, 0)),
            pl.BlockSpec((1, do), lambda i, t: (0, 0)),
        ],
        out_specs=pl.BlockSpec((1, slen // 2, do), lambda i, t: (i, t, 0)),
        out_shape=jax.ShapeDtypeStruct((bsz, slen, do), jnp.float32),
    )(x, w, b.reshape(1, do))


def _corr_body(q_ref, k_ref, v_ref, ch_ref, cl_ref, sh_ref, sl_ref, a_ref,
               corr_ref, vt_ref):
    q = q_ref[...]
    k = k_ref[...]
    v = v_ref[...]
    ch, cl = ch_ref[...], cl_ref[...]
    sh, sl = sh_ref[...], sl_ref[...]
    dn = (((0,), (0,)), ((), ()))
    qa = _mm3_pre(q, ch, cl, dn)
    qb = _mm3_pre(q, sh, sl, dn)
    ka = _mm3_pre(k, ch, cl, dn)
    kb = _mm3_pre(k, sh, sl, dn)
    alpha = a_ref[...]
    pre = (qa * ka + qb * kb) * alpha
    pim = (qa * kb - qb * ka) * alpha
    dnf = (((1,), (1,)), ((), ()))
    corr = _mm3_pre(pre, ch, cl, dnf) - _mm3_pre(pim, sh, sl, dnf)
    corr_ref[...] = corr
    vt_ref[...] = v.T


def _corr_rows(q2, k2, v2, ch, cl, sh, sl, alpha, part=0, nparts=1, nc=256):
    nrow = _NROWS // nparts
    off = part * (nrow // nc)
    grid = (nrow // nc,)
    blk_in = pl.BlockSpec((_L, nc), lambda t: (0, t + off))
    blk_const = lambda shape: pl.BlockSpec(shape, lambda t: (0, 0))
    blk_out = pl.BlockSpec((nc, _L), lambda t: (t, 0))
    return pl.pallas_call(
        _corr_body,
        grid=grid,
        in_specs=[
            blk_in, blk_in, blk_in,
            blk_const((_L, _FPAD)), blk_const((_L, _FPAD)),
            blk_const((_L, _FPAD)), blk_const((_L, _FPAD)),
            blk_const((1, _FPAD)),
        ],
        out_specs=[blk_out, blk_out],
        out_shape=[
            jax.ShapeDtypeStruct((nrow, _L), jnp.float32),
            jax.ShapeDtypeStruct((nrow, _L), jnp.float32),
        ],
    )(q2, k2, v2, ch, cl, sh, sl, alpha)


def _sc_row_compute(crow, v2, acc):
    lane = lax.iota(jnp.int32, _LANES)

    t0v, t0d = plsc.sort_key_val(crow[pl.ds(0, _LANES)], lane)

    def _top16(av, ad, bv, bd):
        bv = lax.rev(bv, (0,))
        bd = lax.rev(bd, (0,))
        keep = av >= bv
        mv = jnp.where(keep, av, bv)
        md = jnp.where(keep, ad, bd)
        nv, nd = plsc.sort_key_val(mv, md)
        return nv, nd

    def merge_pair(c, carry):
        tv, td = carry
        i0 = 2 * c + 1
        av, ad = plsc.sort_key_val(crow[pl.ds(i0 * _LANES, _LANES)],
                                   lane + i0 * _LANES)
        bv, bd = plsc.sort_key_val(crow[pl.ds((i0 + 1) * _LANES, _LANES)],
                                   lane + (i0 + 1) * _LANES)
        pv, pd = _top16(av, ad, bv, bd)
        return _top16(tv, td, pv, pd)

    npair = (_L // _LANES - 1) // 2
    tv, td = lax.fori_loop(0, npair, merge_pair, (t0v, t0d))
    lastc = _L // _LANES - 1
    lv, ld = plsc.sort_key_val(crow[pl.ds(lastc * _LANES, _LANES)],
                               lane + lastc * _LANES)
    tv, td = _top16(tv, td, lv, ld)
    tv, td = plsc.sort_key_val(tv, td, descending=True)

    sel = lane < _TOPK
    vals = jnp.where(sel, tv, jnp.float32(-1e30))
    m = jnp.max(vals)
    e = jnp.where(sel, jnp.exp(vals - m), jnp.float32(0.0))
    w = e / jnp.sum(e)

    ds = [td[i] for i in range(_TOPK)]
    ws = [w[i] for i in range(_TOPK)]

    def agg(c, _):
        base = c * _LANES
        s = ws[0] * v2[pl.ds(base + ds[0], _LANES)]
        for i in range(1, _TOPK):
            s = s + ws[i] * v2[pl.ds(base + ds[i], _LANES)]
        acc[pl.ds(base, _LANES)] = s
        return 0

    lax.fori_loop(0, _L // _LANES, agg, 0)


def _sc_body(nrows, corr_hbm, vt_hbm, out_hbm,
             ca, cb, va, vb, oa, ob, sa, sb, soa, sob):
    nsub = 16
    ncore = 2
    wid = lax.axis_index("s") * ncore + lax.axis_index("c")
    rows_per = nrows // (nsub * ncore)
    base = wid * rows_per

    def start_in(r, cbuf, vbuf, sem):
        pltpu.async_copy(corr_hbm.at[r], cbuf, sem)
        pltpu.async_copy(vt_hbm.at[r], vbuf.at[pl.ds(0, _L)], sem)
        pltpu.async_copy(vt_hbm.at[r], vbuf.at[pl.ds(_L, _L)], sem)

    def wait_in(r, cbuf, vbuf, sem):
        pltpu.make_async_copy(corr_hbm.at[r], cbuf, sem).wait()
        pltpu.make_async_copy(vt_hbm.at[r], vbuf.at[pl.ds(0, _L)], sem).wait()
        pltpu.make_async_copy(vt_hbm.at[r], vbuf.at[pl.ds(_L, _L)], sem).wait()

    start_in(base, ca, va, sa)

    def pair(g, _):
        r0 = base + 2 * g
        r1 = r0 + 1
        start_in(r1, cb, vb, sb)
        wait_in(r0, ca, va, sa)

        @pl.when(g > 0)
        def _():
            pltpu.make_async_copy(oa, out_hbm.at[r0], soa).wait()

        _sc_row_compute(ca, va, oa)
        pltpu.async_copy(oa, out_hbm.at[r0], soa)

        @pl.when(g < rows_per // 2 - 1)
        def _():
            start_in(r0 + 2, ca, va, sa)

        wait_in(r1, cb, vb, sb)

        @pl.when(g > 0)
        def _():
            pltpu.make_async_copy(ob, out_hbm.at[r1], sob).wait()

        _sc_row_compute(cb, vb, ob)
        pltpu.async_copy(ob, out_hbm.at[r1], sob)
        return 0

    lax.fori_loop(0, rows_per // 2, pair, 0)
    pltpu.make_async_copy(oa, out_hbm.at[base], soa).wait()
    pltpu.make_async_copy(ob, out_hbm.at[base], sob).wait()


def _sc_topk_agg(corr_rows, vt_rows):
    nrows = corr_rows.shape[0]
    mesh = plsc.VectorSubcoreMesh(core_axis_name="c", subcore_axis_name="s")
    fn = pl.kernel(
        functools.partial(_sc_body, nrows),
        out_type=jax.ShapeDtypeStruct((nrows, _L), jnp.float32),
        mesh=mesh,
        scratch_types=[
            pltpu.VMEM((_L,), jnp.float32),
            pltpu.VMEM((_L,), jnp.float32),
            pltpu.VMEM((2 * _L,), jnp.float32),
            pltpu.VMEM((2 * _L,), jnp.float32),
            pltpu.VMEM((_L,), jnp.float32),
            pltpu.VMEM((_L,), jnp.float32),
            pltpu.SemaphoreType.DMA,
            pltpu.SemaphoreType.DMA,
            pltpu.SemaphoreType.DMA,
            pltpu.SemaphoreType.DMA,
        ],
        compiler_params=pltpu.CompilerParams(needs_layout_passes=False),
    )
    return fn(corr_rows, vt_rows)


def _out_body(a_ref, w_ref, b_ref, o_ref):
    a = a_ref[...]
    o = lax.dot_general(
        a.astype(jnp.bfloat16), w_ref[...].astype(jnp.bfloat16),
        (((0,), (0,)), ((), ())),
        preferred_element_type=jnp.float32)
    o_ref[...] = o + b_ref[...]


def _out_proj(agg2, w_o, b_o, tb=256):
    nh, d = 16, 512
    out = pl.pallas_call(
        _out_body,
        grid=(nh, _L // tb),
        in_specs=[
            pl.BlockSpec((d, tb), lambda j, t: (j, t)),
            pl.BlockSpec((d, d), lambda j, t: (0, 0)),
            pl.BlockSpec((1, d), lambda j, t: (0, 0)),
        ],
        out_specs=pl.BlockSpec((tb, d), lambda j, t: (t, j)),
        out_shape=jax.ShapeDtypeStruct((_L, nh * d), jnp.float32),
    )(agg2, w_o, b_o.reshape(1, d))
    return out


def _dft_mats():
    i = jnp.arange(_L, dtype=jnp.int32)[:, None]
    f = jnp.arange(_FPAD, dtype=jnp.int32)[None, :]
    phase = jnp.mod(i * f, _L).astype(jnp.float32)
    theta = (2.0 * jnp.pi / _L) * phase
    live = (jnp.arange(_FPAD) < _FREQ)[None, :]
    cm = jnp.where(live, jnp.cos(theta), 0.0)
    sm = jnp.where(live, jnp.sin(theta), 0.0)
    fa = jnp.arange(_FPAD)
    alpha = jnp.where(
        (fa == 0) | (fa == _L // 2), 1.0 / _L,
        jnp.where(fa < _FREQ, 2.0 / _L, 0.0)).astype(jnp.float32)
    return cm, sm, alpha.reshape(1, _FPAD)


def kernel(queries, keys, values, W_q, b_q, W_k, b_k, W_v, b_v, W_o, b_o):
    bsz, slen, d = queries.shape
    qf = _project(queries, W_q, b_q)
    kf = _project(keys, W_k, b_k)
    vf = _project(values, W_v, b_v)
    q2 = qf.reshape(_L, 16 * d)
    k2 = kf.reshape(_L, 16 * d)
    v2 = vf.reshape(_L, 16 * d)
    cm, sm, alpha = _dft_mats()
    ch, cl = _split(cm)
    sh, sl = _split(sm)
    nparts = 4
    parts = []
    for p in range(nparts):
        corr_p, vt_p = _corr_rows(q2, k2, v2, ch, cl, sh, sl, alpha,
                                  part=p, nparts=nparts)
        parts.append(_sc_topk_agg(corr_p, vt_p))
    agg = jnp.concatenate(parts, axis=0)
    out2 = _out_proj(agg, W_o, b_o)
    return out2.reshape(bsz, slen, d)


# speedup vs baseline: 2.1375x; 1.0611x over previous
import functools
import math

import jax
import jax.numpy as jnp
from jax import lax
from jax.experimental import pallas as pl
from jax.experimental.pallas import tpu as pltpu
from jax.experimental.pallas import tpu_sc as plsc

_L = 2048
_FREQ = _L // 2 + 1
_FPAD = 1152
_TOPK = int(2 * math.log(_L))
_NROWS = 8192
_LANES = 16


def _split(x):
    hi = x.astype(jnp.bfloat16)
    lo = (x - hi.astype(jnp.float32)).astype(jnp.bfloat16)
    return hi, lo


def _mm3(a, b, dn):
    ah, al = _split(a)
    bh, bl = _split(b)
    f32 = jnp.float32
    return (lax.dot_general(ah, bh, dn, preferred_element_type=f32)
            + lax.dot_general(ah, bl, dn, preferred_element_type=f32)
            + lax.dot_general(al, bh, dn, preferred_element_type=f32))


def _mm3_pre(a, bh, bl, dn):
    ah, al = _split(a)
    f32 = jnp.float32
    return (lax.dot_general(ah, bh, dn, preferred_element_type=f32)
            + lax.dot_general(ah, bl, dn, preferred_element_type=f32)
            + lax.dot_general(al, bh, dn, preferred_element_type=f32))


def _proj_body(x_ref, w_ref, b_ref, o_ref):
    dn = (((1,), (0,)), ((), ()))
    o_ref[0] = lax.dot_general(
        x_ref[0].astype(jnp.bfloat16), w_ref[...].astype(jnp.bfloat16), dn,
        preferred_element_type=jnp.float32) + b_ref[...]


def _project(x, w, b):
    bsz, slen, d = x.shape
    do = w.shape[1]
    return pl.pallas_call(
        _proj_body,
        grid=(bsz, 2),
        in_specs=[
            pl.BlockSpec((1, slen // 2, d), lambda i, t: (i, t, 0)),
            pl.BlockSpec((d, do), lambda i, t: (0, 0)),
            pl.BlockSpec((1, do), lambda i, t: (0, 0)),
        ],
        out_specs=pl.BlockSpec((1, slen // 2, do), lambda i, t: (i, t, 0)),
        out_shape=jax.ShapeDtypeStruct((bsz, slen, do), jnp.float32),
    )(x, w, b.reshape(1, do))


def _corr_body(q_ref, k_ref, v_ref, ch_ref, cl_ref, sh_ref, sl_ref, a_ref,
               corr_ref, vt_ref):
    q = q_ref[...]
    k = k_ref[...]
    v = v_ref[...]
    ch, cl = ch_ref[...], cl_ref[...]
    sh, sl = sh_ref[...], sl_ref[...]
    dn = (((0,), (0,)), ((), ()))
    qa = _mm3_pre(q, ch, cl, dn)
    qb = _mm3_pre(q, sh, sl, dn)
    ka = _mm3_pre(k, ch, cl, dn)
    kb = _mm3_pre(k, sh, sl, dn)
    alpha = a_ref[...]
    pre = (qa * ka + qb * kb) * alpha
    pim = (qa * kb - qb * ka) * alpha
    dnf = (((1,), (1,)), ((), ()))
    corr = _mm3_pre(pre, ch, cl, dnf) - _mm3_pre(pim, sh, sl, dnf)
    corr_ref[...] = corr
    vt_ref[...] = v.T


def _corr_rows(q2, k2, v2, ch, cl, sh, sl, alpha, part=0, nparts=1, nc=256):
    nrow = _NROWS // nparts
    off = part * (nrow // nc)
    grid = (nrow // nc,)
    blk_in = pl.BlockSpec((_L, nc), lambda t: (0, t + off))
    blk_const = lambda shape: pl.BlockSpec(shape, lambda t: (0, 0))
    blk_out = pl.BlockSpec((nc, _L), lambda t: (t, 0))
    return pl.pallas_call(
        _corr_body,
        grid=grid,
        in_specs=[
            blk_in, blk_in, blk_in,
            blk_const((_L, _FPAD)), blk_const((_L, _FPAD)),
            blk_const((_L, _FPAD)), blk_const((_L, _FPAD)),
            blk_const((1, _FPAD)),
        ],
        out_specs=[blk_out, blk_out],
        out_shape=[
            jax.ShapeDtypeStruct((nrow, _L), jnp.float32),
            jax.ShapeDtypeStruct((nrow, _L), jnp.float32),
        ],
    )(q2, k2, v2, ch, cl, sh, sl, alpha)


def _sc_row_compute(crow, v2, acc):
    lane = lax.iota(jnp.int32, _LANES)

    t0v, t0d = plsc.sort_key_val(crow[pl.ds(0, _LANES)], lane)

    def _top16(av, ad, bv, bd):
        bv = lax.rev(bv, (0,))
        bd = lax.rev(bd, (0,))
        keep = av >= bv
        mv = jnp.where(keep, av, bv)
        md = jnp.where(keep, ad, bd)
        nv, nd = plsc.sort_key_val(mv, md)
        return nv, nd

    def merge_pair(c, carry):
        tv, td = carry
        i0 = 2 * c + 1
        av, ad = plsc.sort_key_val(crow[pl.ds(i0 * _LANES, _LANES)],
                                   lane + i0 * _LANES)
        bv, bd = plsc.sort_key_val(crow[pl.ds((i0 + 1) * _LANES, _LANES)],
                                   lane + (i0 + 1) * _LANES)
        pv, pd = _top16(av, ad, bv, bd)
        return _top16(tv, td, pv, pd)

    npair = (_L // _LANES - 1) // 2
    tv, td = lax.fori_loop(0, npair, merge_pair, (t0v, t0d))
    lastc = _L // _LANES - 1
    lv, ld = plsc.sort_key_val(crow[pl.ds(lastc * _LANES, _LANES)],
                               lane + lastc * _LANES)
    tv, td = _top16(tv, td, lv, ld)
    tv, td = plsc.sort_key_val(tv, td, descending=True)

    sel = lane < _TOPK
    vals = jnp.where(sel, tv, jnp.float32(-1e30))
    m = jnp.max(vals)
    e = jnp.where(sel, jnp.exp(vals - m), jnp.float32(0.0))
    w = e / jnp.sum(e)

    ds = [td[i] for i in range(_TOPK)]
    ws = [w[i] for i in range(_TOPK)]

    def agg(c, _):
        base = c * _LANES
        s = ws[0] * v2[pl.ds(base + ds[0], _LANES)]
        for i in range(1, _TOPK):
            s = s + ws[i] * v2[pl.ds(base + ds[i], _LANES)]
        acc[pl.ds(base, _LANES)] = s
        return 0

    lax.fori_loop(0, _L // _LANES, agg, 0)


def _sc_body(nrows, corr_hbm, vt_hbm, out_hbm,
             ca, cb, va, vb, oa, ob, sa, sb, soa, sob):
    nsub = 16
    ncore = 2
    wid = lax.axis_index("s") * ncore + lax.axis_index("c")
    rows_per = nrows // (nsub * ncore)
    base = wid * rows_per

    def start_in(r, cbuf, vbuf, sem):
        pltpu.async_copy(corr_hbm.at[r], cbuf, sem)
        pltpu.async_copy(vt_hbm.at[r], vbuf.at[pl.ds(0, _L)], sem)
        pltpu.async_copy(vt_hbm.at[r], vbuf.at[pl.ds(_L, _L)], sem)

    def wait_in(r, cbuf, vbuf, sem):
        pltpu.make_async_copy(corr_hbm.at[r], cbuf, sem).wait()
        pltpu.make_async_copy(vt_hbm.at[r], vbuf.at[pl.ds(0, _L)], sem).wait()
        pltpu.make_async_copy(vt_hbm.at[r], vbuf.at[pl.ds(_L, _L)], sem).wait()

    start_in(base, ca, va, sa)

    def pair(g, _):
        r0 = base + 2 * g
        r1 = r0 + 1
        start_in(r1, cb, vb, sb)
        wait_in(r0, ca, va, sa)

        @pl.when(g > 0)
        def _():
            pltpu.make_async_copy(oa, out_hbm.at[r0], soa).wait()

        _sc_row_compute(ca, va, oa)
        pltpu.async_copy(oa, out_hbm.at[r0], soa)

        @pl.when(g < rows_per // 2 - 1)
        def _():
            start_in(r0 + 2, ca, va, sa)

        wait_in(r1, cb, vb, sb)

        @pl.when(g > 0)
        def _():
            pltpu.make_async_copy(ob, out_hbm.at[r1], sob).wait()

        _sc_row_compute(cb, vb, ob)
        pltpu.async_copy(ob, out_hbm.at[r1], sob)
        return 0

    lax.fori_loop(0, rows_per // 2, pair, 0)
    pltpu.make_async_copy(oa, out_hbm.at[base], soa).wait()
    pltpu.make_async_copy(ob, out_hbm.at[base], sob).wait()


def _sc_topk_agg(corr_rows, vt_rows):
    nrows = corr_rows.shape[0]
    mesh = plsc.VectorSubcoreMesh(core_axis_name="c", subcore_axis_name="s")
    fn = pl.kernel(
        functools.partial(_sc_body, nrows),
        out_type=jax.ShapeDtypeStruct((nrows, _L), jnp.float32),
        mesh=mesh,
        scratch_types=[
            pltpu.VMEM((_L,), jnp.float32),
            pltpu.VMEM((_L,), jnp.float32),
            pltpu.VMEM((2 * _L,), jnp.float32),
            pltpu.VMEM((2 * _L,), jnp.float32),
            pltpu.VMEM((_L,), jnp.float32),
            pltpu.VMEM((_L,), jnp.float32),
            pltpu.SemaphoreType.DMA,
            pltpu.SemaphoreType.DMA,
            pltpu.SemaphoreType.DMA,
            pltpu.SemaphoreType.DMA,
        ],
        compiler_params=pltpu.CompilerParams(needs_layout_passes=False),
    )
    return fn(corr_rows, vt_rows)


def _out_body(a_ref, w_ref, b_ref, o_ref):
    a = a_ref[...]
    o = lax.dot_general(
        a.astype(jnp.bfloat16), w_ref[...].astype(jnp.bfloat16),
        (((0,), (0,)), ((), ())),
        preferred_element_type=jnp.float32)
    o_ref[...] = o + b_ref[...]


def _out_proj(agg2, w_o, b_o, tb=256):
    nh, d = 16, 512
    out = pl.pallas_call(
        _out_body,
        grid=(nh, _L // tb),
        in_specs=[
            pl.BlockSpec((d, tb), lambda j, t: (j, t)),
            pl.BlockSpec((d, d), lambda j, t: (0, 0)),
            pl.BlockSpec((1, d), lambda j, t: (0, 0)),
        ],
        out_specs=pl.BlockSpec((tb, d), lambda j, t: (t, j)),
        out_shape=jax.ShapeDtypeStruct((_L, nh * d), jnp.float32),
    )(agg2, w_o, b_o.reshape(1, d))
    return out


def _dft_mats():
    i = jnp.arange(_L, dtype=jnp.int32)[:, None]
    f = jnp.arange(_FPAD, dtype=jnp.int32)[None, :]
    phase = jnp.mod(i * f, _L).astype(jnp.float32)
    theta = (2.0 * jnp.pi / _L) * phase
    live = (jnp.arange(_FPAD) < _FREQ)[None, :]
    cm = jnp.where(live, jnp.cos(theta), 0.0)
    sm = jnp.where(live, jnp.sin(theta), 0.0)
    fa = jnp.arange(_FPAD)
    alpha = jnp.where(
        (fa == 0) | (fa == _L // 2), 1.0 / _L,
        jnp.where(fa < _FREQ, 2.0 / _L, 0.0)).astype(jnp.float32)
    return cm, sm, alpha.reshape(1, _FPAD)


def kernel(queries, keys, values, W_q, b_q, W_k, b_k, W_v, b_v, W_o, b_o):
    bsz, slen, d = queries.shape
    qf = _project(queries, W_q, b_q)
    kf = _project(keys, W_k, b_k)
    vf = _project(values, W_v, b_v)
    q2 = qf.reshape(_L, 16 * d)
    k2 = kf.reshape(_L, 16 * d)
    v2 = vf.reshape(_L, 16 * d)
    cm, sm, alpha = _dft_mats()
    ch, cl = _split(cm)
    sh, sl = _split(sm)
    nparts = 8
    parts = []
    for p in range(nparts):
        corr_p, vt_p = _corr_rows(q2, k2, v2, ch, cl, sh, sl, alpha,
                                  part=p, nparts=nparts)
        parts.append(_sc_topk_agg(corr_p, vt_p))
    agg = jnp.concatenate(parts, axis=0)
    out2 = _out_proj(agg, W_o, b_o)
    return out2.reshape(bsz, slen, d)
